# baseline (speedup 1.0000x reference)
import functools

import jax
import jax.numpy as jnp
from jax import lax
from jax.experimental import pallas as pl
from jax.experimental.pallas import tpu as pltpu
from jax.experimental.pallas import tpu_sc as plsc

N = 10000
T = 4
D = 128
H = 128
E = 320000
OUT = 1

NC = 2
NS = 16
NW = NC * NS

PAD = 16
CH = 128
NCH = 80
HCH = NCH // 2
EPW = NCH * CH
EPAD = NW * EPW
NPE = EPAD - E

ROWS_PT = 624
ROWS_TAIL = N - NS * ROWS_PT

DEG_PT = 2496
DEG_TAIL = T * N - NS * DEG_PT


def _zero_vmem(ref, nwords):
  z = jnp.zeros((16,), dtype=jnp.float32)

  def body(i, _):
    ref[pl.ds(i * 16, 16)] = z
    return 0

  lax.fori_loop(0, nwords // 16, body, 0)


def _zero_vmem2d(ref, nrows, ncols):
  z = jnp.zeros((16,), dtype=jnp.float32)
  cchunks = ncols // 16

  def body(i, _):
    r = i // cchunks
    cs = (i % cchunks) * 16
    ref[r, pl.ds(cs, 16)] = z
    return 0

  lax.fori_loop(0, nrows * cchunks, body, 0)


def _deg_body(didx_hbm, out_hbm, idx_v, ones_v, zv, acc, sem):
  c = lax.axis_index("c")
  s = lax.axis_index("s")
  wid = c * NS + s

  for i in range(CH // 16):
    ones_v[pl.ds(i * 16, 16)] = jnp.ones((16,), dtype=jnp.float32)
  _zero_vmem(zv, DEG_PT)

  pltpu.sync_copy(zv, acc.at[pl.ds(s * DEG_PT, DEG_PT)])

  @pl.when(s == 0)
  def _():
    pltpu.sync_copy(zv.at[pl.ds(0, DEG_TAIL + PAD)],
                    acc.at[pl.ds(NS * DEG_PT, DEG_TAIL + PAD)])

  plsc.subcore_barrier()

  K = 8

  for t in range(T):
    pltpu.sync_copy(didx_hbm.at[t, wid], idx_v)

    def body(g, _):
      for j in range(K):
        pltpu.async_copy(ones_v, acc.at[idx_v.at[g * K + j]], sem, add=True)
      for j in range(K):
        pltpu.make_async_copy(ones_v, acc.at[idx_v.at[g * K + j]], sem).wait()
      return 0

    lax.fori_loop(0, NCH // K, body, 0)

  plsc.subcore_barrier()

  pltpu.sync_copy(acc.at[pl.ds(s * DEG_PT, DEG_PT)], zv)
  pltpu.sync_copy(zv, out_hbm.at[pl.ds(c * T * N + s * DEG_PT, DEG_PT)])

  @pl.when(s == 0)
  def _():
    pltpu.sync_copy(acc.at[pl.ds(NS * DEG_PT, DEG_TAIL)],
                    zv.at[pl.ds(0, DEG_TAIL)])
    pltpu.sync_copy(zv.at[pl.ds(0, DEG_TAIL)],
                    out_hbm.at[pl.ds(c * T * N + NS * DEG_PT, DEG_TAIL)])


def _edge_body(y_hbm, src_hbm, dst_hbm, out_hbm, sidx, didx, rows0, rows1,
               acc, semA, semB, semA2, semB2):
  c = lax.axis_index("c")
  s = lax.axis_index("s")
  wid = c * NS + s

  for t in range(T):
    _zero_vmem2d(rows0, CH, H)
    for j in range(ROWS_PT // CH):
      pltpu.sync_copy(rows0, acc.at[pl.ds(s * ROWS_PT + j * CH, CH)])
    rem = ROWS_PT - (ROWS_PT // CH) * CH
    pltpu.sync_copy(rows0.at[pl.ds(0, rem)],
                    acc.at[pl.ds(s * ROWS_PT + (ROWS_PT // CH) * CH, rem)])

    @pl.when(s == 0)
    def _():
      pltpu.sync_copy(rows0.at[pl.ds(0, ROWS_TAIL + PAD)],
                      acc.at[pl.ds(NS * ROWS_PT, ROWS_TAIL + PAD)])

    plsc.subcore_barrier()

    for half in range(2):
      pltpu.sync_copy(src_hbm.at[t, wid, pl.ds(half * HCH, HCH)], sidx)
      pltpu.sync_copy(dst_hbm.at[t, wid, pl.ds(half * HCH, HCH)], didx)

      HC = CH // 2

      def gather(i, buf, s1, s2):
        pltpu.async_copy(y_hbm.at[sidx.at[i, pl.ds(0, HC)]],
                         buf.at[pl.ds(0, HC)], s1)
        pltpu.async_copy(y_hbm.at[sidx.at[i, pl.ds(HC, HC)]],
                         buf.at[pl.ds(HC, HC)], s2)

      def gwait(i, buf, s1, s2):
        pltpu.make_async_copy(y_hbm.at[sidx.at[i, pl.ds(0, HC)]],
                              buf.at[pl.ds(0, HC)], s1).wait()
        pltpu.make_async_copy(y_hbm.at[sidx.at[i, pl.ds(HC, HC)]],
                              buf.at[pl.ds(HC, HC)], s2).wait()

      gather(0, rows0, semA, semA2)

      def body(g, _):
        i0 = 2 * g
        gather(i0 + 1, rows1, semB, semB2)
        gwait(i0, rows0, semA, semA2)
        pltpu.sync_copy(rows0, acc.at[didx.at[i0]], add=True)

        @pl.when(g < HCH // 2 - 1)
        def _():
          gather(i0 + 2, rows0, semA, semA2)

        gwait(i0 + 1, rows1, semB, semB2)
        pltpu.sync_copy(rows1, acc.at[didx.at[i0 + 1]], add=True)
        return 0

      lax.fori_loop(0, HCH // 2, body, 0)

    plsc.subcore_barrier()

    obase = c * T * N + t * N
    for j in range(ROWS_PT // CH):
      pltpu.sync_copy(acc.at[pl.ds(s * ROWS_PT + j * CH, CH)], rows0)
      pltpu.sync_copy(rows0,
                      out_hbm.at[pl.ds(obase + s * ROWS_PT + j * CH, CH)])
    pltpu.sync_copy(acc.at[pl.ds(s * ROWS_PT + (ROWS_PT // CH) * CH, rem)],
                    rows0.at[pl.ds(0, rem)])
    pltpu.sync_copy(rows0.at[pl.ds(0, rem)],
                    out_hbm.at[pl.ds(obase + s * ROWS_PT + (ROWS_PT // CH) * CH,
                                     rem)])

    @pl.when(s == 0)
    def _():
      pltpu.sync_copy(acc.at[pl.ds(NS * ROWS_PT, ROWS_TAIL)],
                      rows1.at[pl.ds(0, ROWS_TAIL)])
      pltpu.sync_copy(rows1.at[pl.ds(0, ROWS_TAIL)],
                      out_hbm.at[pl.ds(obase + NS * ROWS_PT, ROWS_TAIL)])

    plsc.subcore_barrier()


BLK_A = 2000
GRID_A = (T * N) // BLK_A


def _tc_scale_body(x_ref, wt_ref, deg_ref, y_ref):
  deg = deg_ref[:, 0] + deg_ref[:, 1] + 1.0
  dinv = lax.rsqrt(deg)
  xw = jnp.dot(x_ref[...], wt_ref[...], preferred_element_type=jnp.float32)
  y_ref[...] = xw * dinv[:, None]


_tc_scale = pl.pallas_call(
    _tc_scale_body,
    grid=(GRID_A,),
    in_specs=[
        pl.BlockSpec((BLK_A, D), lambda i: (i, 0)),
        pl.BlockSpec((D, H), lambda i: (0, 0)),
        pl.BlockSpec((BLK_A, NC), lambda i: (i, 0)),
    ],
    out_specs=pl.BlockSpec((BLK_A, H), lambda i: (i, 0)),
    out_shape=jax.ShapeDtypeStruct((T * N + PAD, H), jnp.float32),
)


BLK_C = 1000
GRID_C = N // BLK_C


def _tc_gru_body(p_ref, y0_ref, y1_ref, y2_ref, y3_ref, deg_ref, gcn_b_ref,
                 wih_ref, whh_ref, bih_ref, bhh_ref, hw_ref, hb_ref,
                 risk_ref, h_ref):
  h = jnp.zeros((BLK_C, H), dtype=jnp.float32)
  bih = bih_ref[...]
  bhh = bhh_ref[...]
  y_refs = (y0_ref, y1_ref, y2_ref, y3_ref)
  for t in range(T):
    deg = deg_ref[t, :, 0] + deg_ref[t, :, 1] + 1.0
    dinv = lax.rsqrt(deg)
    agg = p_ref[0, t] + p_ref[1, t] + y_refs[t][...]
    xt = jax.nn.relu(agg * dinv[:, None] + gcn_b_ref[...])
    gi = jnp.dot(xt.astype(jnp.bfloat16), wih_ref[...],
                 preferred_element_type=jnp.float32) + bih
    gh = jnp.dot(h.astype(jnp.bfloat16), whh_ref[...],
                 preferred_element_type=jnp.float32) + bhh
    r = jax.nn.sigmoid(gi[:, 0:H] + gh[:, 0:H])
    z = jax.nn.sigmoid(gi[:, H:2 * H] + gh[:, H:2 * H])
    n = jnp.tanh(gi[:, 2 * H:] + r * gh[:, 2 * H:])
    h = (1.0 - z) * n + z * h
  risk = jax.nn.sigmoid(
      jnp.sum(h * hw_ref[...], axis=1, keepdims=True) + hb_ref[0, 0])
  risk_ref[...] = risk
  h_ref[...] = h


_tc_gru = pl.pallas_call(
    _tc_gru_body,
    grid=(GRID_C,),
    in_specs=[
        pl.BlockSpec((NC, T, BLK_C, H), lambda i: (0, 0, i, 0)),
        pl.BlockSpec((BLK_C, H), lambda i: (0 * (N // BLK_C) + i, 0)),
        pl.BlockSpec((BLK_C, H), lambda i: (1 * (N // BLK_C) + i, 0)),
        pl.BlockSpec((BLK_C, H), lambda i: (2 * (N // BLK_C) + i, 0)),
        pl.BlockSpec((BLK_C, H), lambda i: (3 * (N // BLK_C) + i, 0)),
        pl.BlockSpec((T, BLK_C, NC), lambda i: (0, i, 0)),
        pl.BlockSpec((1, H), lambda i: (0, 0)),
        pl.BlockSpec((H, 3 * H), lambda i: (0, 0)),
        pl.BlockSpec((H, 3 * H), lambda i: (0, 0)),
        pl.BlockSpec((1, 3 * H), lambda i: (0, 0)),
        pl.BlockSpec((1, 3 * H), lambda i: (0, 0)),
        pl.BlockSpec((1, H), lambda i: (0, 0)),
        pl.BlockSpec((1, 1), lambda i: (0, 0)),
    ],
    out_specs=[
        pl.BlockSpec((BLK_C, 1), lambda i: (i, 0)),
        pl.BlockSpec((BLK_C, H), lambda i: (i, 0)),
    ],
    out_shape=[
        jax.ShapeDtypeStruct((N, OUT), jnp.float32),
        jax.ShapeDtypeStruct((N, H), jnp.float32),
    ],
)


@functools.lru_cache(maxsize=1)
def _build_sc_kernels():
  mesh = plsc.VectorSubcoreMesh(
      core_axis_name="c", subcore_axis_name="s", num_cores=NC, num_subcores=NS
  )
  deg_kernel = pl.kernel(
      _deg_body,
      out_type=jax.ShapeDtypeStruct((NC * T * N,), jnp.float32),
      mesh=mesh,
      scratch_types=[
          pltpu.VMEM((NCH, CH), jnp.int32),
          pltpu.VMEM((CH,), jnp.float32),
          pltpu.VMEM((DEG_PT,), jnp.float32),
          pltpu.VMEM_SHARED((T * N + PAD,), jnp.float32),
          pltpu.SemaphoreType.DMA,
      ],
  )
  edge_kernel = pl.kernel(
      _edge_body,
      out_type=jax.ShapeDtypeStruct((NC * T * N, H), jnp.float32),
      mesh=mesh,
      scratch_types=[
          pltpu.VMEM((HCH, CH), jnp.int32),
          pltpu.VMEM((HCH, CH), jnp.int32),
          pltpu.VMEM((CH, H), jnp.float32),
          pltpu.VMEM((CH, H), jnp.float32),
          pltpu.VMEM_SHARED((N + PAD, H), jnp.float32),
          pltpu.SemaphoreType.DMA,
          pltpu.SemaphoreType.DMA,
          pltpu.SemaphoreType.DMA,
          pltpu.SemaphoreType.DMA,
      ],
  )
  return deg_kernel, edge_kernel


def _prep_indices(edge_index_seq):
  toff = (jnp.arange(T, dtype=jnp.int32) * N)[:, None]
  padv = (jnp.arange(NPE, dtype=jnp.int32) % PAD)[None, :]
  padv = jnp.broadcast_to(padv, (T, NPE))

  src = jnp.concatenate([edge_index_seq[:, 0, :] + toff, padv + T * N], axis=1)
  dst = jnp.concatenate([edge_index_seq[:, 1, :], padv + N], axis=1)
  dd = jnp.concatenate([edge_index_seq[:, 1, :] + toff, padv + T * N], axis=1)

  src4 = src.reshape(T, NW, NCH, CH)
  dst4 = dst.reshape(T, NW, NCH, CH)
  didx = dd.reshape(T, NW, NCH, CH)
  return src4, dst4, didx


def kernel(x_seq, edge_index_seq, gcn_w, gcn_b, w_ih, w_hh, b_ih, b_hh,
           head_w, head_b):
  deg_kernel, edge_kernel = _build_sc_kernels()
  src4, dst4, didx = _prep_indices(edge_index_seq)

  deg_p = deg_kernel(didx)
  deg2 = deg_p.reshape(NC, T * N).T

  x_flat = x_seq.reshape(T * N, D)
  y = _tc_scale(x_flat, gcn_w.T, deg2)

  p = edge_kernel(y, src4, dst4)

  risk, h_final = _tc_gru(
      p.reshape(NC, T, N, H),
      y, y, y, y,
      deg_p.reshape(NC, T, N).transpose(1, 2, 0),
      gcn_b.reshape(1, H),
      w_ih.T.astype(jnp.bfloat16),
      w_hh.T.astype(jnp.bfloat16),
      b_ih.reshape(1, 3 * H),
      b_hh.reshape(1, 3 * H),
      head_w.reshape(1, H),
      head_b.reshape(1, 1),
  )
  return risk, h_final


# speedup vs baseline: 1.0138x; 1.0138x over previous
import functools

import jax
import jax.numpy as jnp
from jax import lax
from jax.experimental import pallas as pl
from jax.experimental.pallas import tpu as pltpu
from jax.experimental.pallas import tpu_sc as plsc

N = 10000
T = 4
D = 128
H = 128
E = 320000
OUT = 1

NC = 2
NS = 16
NW = NC * NS

PAD = 16
CH = 128
NCH = 80
HCH = NCH // 2
TCH = E // CH
TAILW = NW - 1
TCHW = TCH - TAILW * NCH

ROWS_PT = 624
ROWS_TAIL = N - NS * ROWS_PT

DEG_PT = 2496
DEG_TAIL = T * N - NS * DEG_PT


def _zero_vmem(ref, nwords):
  z = jnp.zeros((16,), dtype=jnp.float32)

  def body(i, _):
    ref[pl.ds(i * 16, 16)] = z
    return 0

  lax.fori_loop(0, nwords // 16, body, 0)


def _zero_vmem2d(ref, nrows, ncols):
  z = jnp.zeros((16,), dtype=jnp.float32)
  cchunks = ncols // 16

  def body(i, _):
    r = i // cchunks
    cs = (i % cchunks) * 16
    ref[r, pl.ds(cs, 16)] = z
    return 0

  lax.fori_loop(0, nrows * cchunks, body, 0)


def _deg_body(didx_hbm, out_hbm, idx_v, ones_v, zv, acc, sem):
  c = lax.axis_index("c")
  s = lax.axis_index("s")
  wid = c * NS + s

  for i in range(CH // 16):
    ones_v[pl.ds(i * 16, 16)] = jnp.ones((16,), dtype=jnp.float32)
  _zero_vmem(zv, DEG_PT)

  pltpu.sync_copy(zv, acc.at[pl.ds(s * DEG_PT, DEG_PT)])

  @pl.when(s == 0)
  def _():
    pltpu.sync_copy(zv.at[pl.ds(0, DEG_TAIL + PAD)],
                    acc.at[pl.ds(NS * DEG_PT, DEG_TAIL + PAD)])

  plsc.subcore_barrier()

  K = 4

  def scat(nch):
    def body(g, _):
      for j in range(K):
        pltpu.async_copy(ones_v, acc.at[idx_v.at[g * K + j]], sem, add=True)
      for j in range(K):
        pltpu.make_async_copy(ones_v, acc.at[idx_v.at[g * K + j]], sem).wait()
      return 0

    lax.fori_loop(0, nch // K, body, 0)

  for t in range(T):
    @pl.when(wid < TAILW)
    def _():
      pltpu.sync_copy(didx_hbm.at[t, pl.ds(wid * NCH, NCH)], idx_v)
      scat(NCH)

    @pl.when(wid == TAILW)
    def _():
      pltpu.sync_copy(didx_hbm.at[t, pl.ds(TAILW * NCH, TCHW)],
                      idx_v.at[pl.ds(0, TCHW)])
      scat(TCHW)

  plsc.subcore_barrier()

  pltpu.sync_copy(acc.at[pl.ds(s * DEG_PT, DEG_PT)], zv)
  pltpu.sync_copy(zv, out_hbm.at[pl.ds(c * T * N + s * DEG_PT, DEG_PT)])

  @pl.when(s == 0)
  def _():
    pltpu.sync_copy(acc.at[pl.ds(NS * DEG_PT, DEG_TAIL)],
                    zv.at[pl.ds(0, DEG_TAIL)])
    pltpu.sync_copy(zv.at[pl.ds(0, DEG_TAIL)],
                    out_hbm.at[pl.ds(c * T * N + NS * DEG_PT, DEG_TAIL)])


def _edge_body(y_hbm, src_hbm, dst_hbm, out_hbm, sidx, didx, rows0, rows1,
               acc, semA, semB):
  c = lax.axis_index("c")
  s = lax.axis_index("s")
  wid = c * NS + s

  for t in range(T):
    _zero_vmem2d(rows0, CH, H)
    for j in range(ROWS_PT // CH):
      pltpu.sync_copy(rows0, acc.at[pl.ds(s * ROWS_PT + j * CH, CH)])
    rem = ROWS_PT - (ROWS_PT // CH) * CH
    pltpu.sync_copy(rows0.at[pl.ds(0, rem)],
                    acc.at[pl.ds(s * ROWS_PT + (ROWS_PT // CH) * CH, rem)])

    @pl.when(s == 0)
    def _():
      pltpu.sync_copy(rows0.at[pl.ds(0, ROWS_TAIL + PAD)],
                      acc.at[pl.ds(NS * ROWS_PT, ROWS_TAIL + PAD)])

    plsc.subcore_barrier()

    def pipeline(npairs):
      pltpu.async_copy(y_hbm.at[sidx.at[0]], rows0, semA)

      def body(g, _):
        i0 = 2 * g
        pltpu.async_copy(y_hbm.at[sidx.at[i0 + 1]], rows1, semB)
        pltpu.make_async_copy(y_hbm.at[sidx.at[i0]], rows0, semA).wait()
        pltpu.sync_copy(rows0, acc.at[didx.at[i0]], add=True)

        @pl.when(g < npairs - 1)
        def _():
          pltpu.async_copy(y_hbm.at[sidx.at[i0 + 2]], rows0, semA)

        pltpu.make_async_copy(y_hbm.at[sidx.at[i0 + 1]], rows1, semB).wait()
        pltpu.sync_copy(rows1, acc.at[didx.at[i0 + 1]], add=True)
        return 0

      lax.fori_loop(0, npairs, body, 0)

    @pl.when(wid < TAILW)
    def _():
      for half in range(2):
        base = wid * NCH + half * HCH
        pltpu.sync_copy(src_hbm.at[t, pl.ds(base, HCH)], sidx)
        pltpu.sync_copy(dst_hbm.at[t, pl.ds(base, HCH)], didx)
        pipeline(HCH // 2)

    @pl.when(wid == TAILW)
    def _():
      base = TAILW * NCH
      pltpu.sync_copy(src_hbm.at[t, pl.ds(base, TCHW)],
                      sidx.at[pl.ds(0, TCHW)])
      pltpu.sync_copy(dst_hbm.at[t, pl.ds(base, TCHW)],
                      didx.at[pl.ds(0, TCHW)])
      pipeline(TCHW // 2)

    plsc.subcore_barrier()

    obase = c * T * N + t * N
    for j in range(ROWS_PT // CH):
      pltpu.sync_copy(acc.at[pl.ds(s * ROWS_PT + j * CH, CH)], rows0)
      pltpu.sync_copy(rows0,
                      out_hbm.at[pl.ds(obase + s * ROWS_PT + j * CH, CH)])
    pltpu.sync_copy(acc.at[pl.ds(s * ROWS_PT + (ROWS_PT // CH) * CH, rem)],
                    rows0.at[pl.ds(0, rem)])
    pltpu.sync_copy(rows0.at[pl.ds(0, rem)],
                    out_hbm.at[pl.ds(obase + s * ROWS_PT + (ROWS_PT // CH) * CH,
                                     rem)])

    @pl.when(s == 0)
    def _():
      pltpu.sync_copy(acc.at[pl.ds(NS * ROWS_PT, ROWS_TAIL)],
                      rows1.at[pl.ds(0, ROWS_TAIL)])
      pltpu.sync_copy(rows1.at[pl.ds(0, ROWS_TAIL)],
                      out_hbm.at[pl.ds(obase + NS * ROWS_PT, ROWS_TAIL)])

    plsc.subcore_barrier()


BLK_A = 2000
GRID_A = (T * N) // BLK_A


def _tc_scale_body(x_ref, wt_ref, deg_ref, y_ref):
  deg = deg_ref[:, 0] + deg_ref[:, 1] + 1.0
  dinv = lax.rsqrt(deg)
  xw = jnp.dot(x_ref[...], wt_ref[...], preferred_element_type=jnp.float32)
  y_ref[...] = xw * dinv[:, None]


_tc_scale = pl.pallas_call(
    _tc_scale_body,
    grid=(GRID_A,),
    in_specs=[
        pl.BlockSpec((BLK_A, D), lambda i: (i, 0)),
        pl.BlockSpec((D, H), lambda i: (0, 0)),
        pl.BlockSpec((BLK_A, NC), lambda i: (i, 0)),
    ],
    out_specs=pl.BlockSpec((BLK_A, H), lambda i: (i, 0)),
    out_shape=jax.ShapeDtypeStruct((T * N + PAD, H), jnp.float32),
)


BLK_C = 1000
GRID_C = N // BLK_C


def _tc_gru_body(p_ref, y0_ref, y1_ref, y2_ref, y3_ref, deg_ref, gcn_b_ref,
                 wih_ref, whh_ref, bih_ref, bhh_ref, hw_ref, hb_ref,
                 risk_ref, h_ref):
  h = jnp.zeros((BLK_C, H), dtype=jnp.float32)
  bih = bih_ref[...]
  bhh = bhh_ref[...]
  y_refs = (y0_ref, y1_ref, y2_ref, y3_ref)
  for t in range(T):
    deg = deg_ref[t, :, 0] + deg_ref[t, :, 1] + 1.0
    dinv = lax.rsqrt(deg)
    agg = p_ref[0, t] + p_ref[1, t] + y_refs[t][...]
    xt = jax.nn.relu(agg * dinv[:, None] + gcn_b_ref[...])
    gi = jnp.dot(xt.astype(jnp.bfloat16), wih_ref[...],
                 preferred_element_type=jnp.float32) + bih
    gh = jnp.dot(h.astype(jnp.bfloat16), whh_ref[...],
                 preferred_element_type=jnp.float32) + bhh
    r = jax.nn.sigmoid(gi[:, 0:H] + gh[:, 0:H])
    z = jax.nn.sigmoid(gi[:, H:2 * H] + gh[:, H:2 * H])
    n = jnp.tanh(gi[:, 2 * H:] + r * gh[:, 2 * H:])
    h = (1.0 - z) * n + z * h
  risk = jax.nn.sigmoid(
      jnp.sum(h * hw_ref[...], axis=1, keepdims=True) + hb_ref[0, 0])
  risk_ref[...] = risk
  h_ref[...] = h


_tc_gru = pl.pallas_call(
    _tc_gru_body,
    grid=(GRID_C,),
    in_specs=[
        pl.BlockSpec((NC, T, BLK_C, H), lambda i: (0, 0, i, 0)),
        pl.BlockSpec((BLK_C, H), lambda i: (0 * (N // BLK_C) + i, 0)),
        pl.BlockSpec((BLK_C, H), lambda i: (1 * (N // BLK_C) + i, 0)),
        pl.BlockSpec((BLK_C, H), lambda i: (2 * (N // BLK_C) + i, 0)),
        pl.BlockSpec((BLK_C, H), lambda i: (3 * (N // BLK_C) + i, 0)),
        pl.BlockSpec((T, BLK_C, NC), lambda i: (0, i, 0)),
        pl.BlockSpec((1, H), lambda i: (0, 0)),
        pl.BlockSpec((H, 3 * H), lambda i: (0, 0)),
        pl.BlockSpec((H, 3 * H), lambda i: (0, 0)),
        pl.BlockSpec((1, 3 * H), lambda i: (0, 0)),
        pl.BlockSpec((1, 3 * H), lambda i: (0, 0)),
        pl.BlockSpec((1, H), lambda i: (0, 0)),
        pl.BlockSpec((1, 1), lambda i: (0, 0)),
    ],
    out_specs=[
        pl.BlockSpec((BLK_C, 1), lambda i: (i, 0)),
        pl.BlockSpec((BLK_C, H), lambda i: (i, 0)),
    ],
    out_shape=[
        jax.ShapeDtypeStruct((N, OUT), jnp.float32),
        jax.ShapeDtypeStruct((N, H), jnp.float32),
    ],
)


@functools.lru_cache(maxsize=1)
def _build_sc_kernels():
  mesh = plsc.VectorSubcoreMesh(
      core_axis_name="c", subcore_axis_name="s", num_cores=NC, num_subcores=NS
  )
  deg_kernel = pl.kernel(
      _deg_body,
      out_type=jax.ShapeDtypeStruct((NC * T * N,), jnp.float32),
      mesh=mesh,
      scratch_types=[
          pltpu.VMEM((NCH, CH), jnp.int32),
          pltpu.VMEM((CH,), jnp.float32),
          pltpu.VMEM((DEG_PT,), jnp.float32),
          pltpu.VMEM_SHARED((T * N + PAD,), jnp.float32),
          pltpu.SemaphoreType.DMA,
      ],
  )
  edge_kernel = pl.kernel(
      _edge_body,
      out_type=jax.ShapeDtypeStruct((NC * T * N, H), jnp.float32),
      mesh=mesh,
      scratch_types=[
          pltpu.VMEM((HCH, CH), jnp.int32),
          pltpu.VMEM((HCH, CH), jnp.int32),
          pltpu.VMEM((CH, H), jnp.float32),
          pltpu.VMEM((CH, H), jnp.float32),
          pltpu.VMEM_SHARED((N + PAD, H), jnp.float32),
          pltpu.SemaphoreType.DMA,
          pltpu.SemaphoreType.DMA,
      ],
  )
  return deg_kernel, edge_kernel


def _prep_indices(edge_index_seq):
  toff = (jnp.arange(T, dtype=jnp.int32) * N)[:, None]
  src3 = (edge_index_seq[:, 0, :] + toff).reshape(T, TCH, CH)
  dst3 = edge_index_seq[:, 1, :].reshape(T, TCH, CH)
  didx = (edge_index_seq[:, 1, :] + toff).reshape(T, TCH, CH)
  return src3, dst3, didx


def kernel(x_seq, edge_index_seq, gcn_w, gcn_b, w_ih, w_hh, b_ih, b_hh,
           head_w, head_b):
  deg_kernel, edge_kernel = _build_sc_kernels()
  src3, dst3, didx = _prep_indices(edge_index_seq)

  deg_p = deg_kernel(didx)
  deg2 = deg_p.reshape(NC, T * N).T

  x_flat = x_seq.reshape(T * N, D)
  y = _tc_scale(x_flat, gcn_w.T, deg2)

  p = edge_kernel(y, src3, dst3)

  risk, h_final = _tc_gru(
      p.reshape(NC, T, N, H),
      y, y, y, y,
      deg_p.reshape(NC, T, N).transpose(1, 2, 0),
      gcn_b.reshape(1, H),
      w_ih.T.astype(jnp.bfloat16),
      w_hh.T.astype(jnp.bfloat16),
      b_ih.reshape(1, 3 * H),
      b_hh.reshape(1, 3 * H),
      head_w.reshape(1, H),
      head_b.reshape(1, 1),
  )
  return risk, h_final


# speedup vs baseline: 1.0198x; 1.0059x over previous
import functools

import jax
import jax.numpy as jnp
from jax import lax
from jax.experimental import pallas as pl
from jax.experimental.pallas import tpu as pltpu
from jax.experimental.pallas import tpu_sc as plsc

N = 10000
T = 4
D = 128
H = 128
E = 320000
OUT = 1

NC = 2
NS = 16
NW = NC * NS

PAD = 16
CH = 128
NCH = 80
HCH = NCH // 2
TCH = E // CH
TAILW = NW - 1
TCHW = TCH - TAILW * NCH

ROWS_PT = 624
ROWS_TAIL = N - NS * ROWS_PT

DEG_PT = 2496
DEG_TAIL = T * N - NS * DEG_PT


def _zero_vmem(ref, nwords):
  z = jnp.zeros((16,), dtype=jnp.float32)

  def body(i, _):
    ref[pl.ds(i * 16, 16)] = z
    return 0

  lax.fori_loop(0, nwords // 16, body, 0)


def _zero_vmem2d(ref, nrows, ncols):
  z = jnp.zeros((16,), dtype=jnp.float32)
  cchunks = ncols // 16

  def body(i, _):
    r = i // cchunks
    cs = (i % cchunks) * 16
    ref[r, pl.ds(cs, 16)] = z
    return 0

  lax.fori_loop(0, nrows * cchunks, body, 0)


def _deg_body(didx_hbm, out_hbm, idx_v, ones_v, zv, acc, sem):
  c = lax.axis_index("c")
  s = lax.axis_index("s")
  wid = c * NS + s

  for i in range(CH // 16):
    ones_v[pl.ds(i * 16, 16)] = jnp.ones((16,), dtype=jnp.float32)
  _zero_vmem(zv, DEG_PT)

  pltpu.sync_copy(zv, acc.at[pl.ds(s * DEG_PT, DEG_PT)])

  @pl.when(s == 0)
  def _():
    pltpu.sync_copy(zv.at[pl.ds(0, DEG_TAIL + PAD)],
                    acc.at[pl.ds(NS * DEG_PT, DEG_TAIL + PAD)])

  plsc.subcore_barrier()

  K = 4

  def scat(nch):
    def body(g, _):
      for j in range(K):
        pltpu.async_copy(ones_v, acc.at[idx_v.at[g * K + j]], sem, add=True)
      for j in range(K):
        pltpu.make_async_copy(ones_v, acc.at[idx_v.at[g * K + j]], sem).wait()
      return 0

    lax.fori_loop(0, nch // K, body, 0)

  for t in range(T):
    @pl.when(wid < TAILW)
    def _():
      pltpu.sync_copy(didx_hbm.at[t, pl.ds(wid * NCH, NCH)], idx_v)
      scat(NCH)

    @pl.when(wid == TAILW)
    def _():
      pltpu.sync_copy(didx_hbm.at[t, pl.ds(TAILW * NCH, TCHW)],
                      idx_v.at[pl.ds(0, TCHW)])
      scat(TCHW)

  plsc.subcore_barrier()

  pltpu.sync_copy(acc.at[pl.ds(s * DEG_PT, DEG_PT)], zv)
  pltpu.sync_copy(zv, out_hbm.at[pl.ds(c * T * N + s * DEG_PT, DEG_PT)])

  @pl.when(s == 0)
  def _():
    pltpu.sync_copy(acc.at[pl.ds(NS * DEG_PT, DEG_TAIL)],
                    zv.at[pl.ds(0, DEG_TAIL)])
    pltpu.sync_copy(zv.at[pl.ds(0, DEG_TAIL)],
                    out_hbm.at[pl.ds(c * T * N + NS * DEG_PT, DEG_TAIL)])


def _edge_body(y_hbm, src_hbm, dst_hbm, out_hbm, sidx, didx, rows0, rows1,
               acc, semA, semB):
  c = lax.axis_index("c")
  s = lax.axis_index("s")
  wid = c * NS + s

  for t in range(T):
    _zero_vmem2d(rows0, CH, H)
    for j in range(ROWS_PT // CH):
      pltpu.sync_copy(rows0, acc.at[pl.ds(s * ROWS_PT + j * CH, CH)])
    rem = ROWS_PT - (ROWS_PT // CH) * CH
    pltpu.sync_copy(rows0.at[pl.ds(0, rem)],
                    acc.at[pl.ds(s * ROWS_PT + (ROWS_PT // CH) * CH, rem)])

    @pl.when(s == 0)
    def _():
      pltpu.sync_copy(rows0.at[pl.ds(0, ROWS_TAIL + PAD)],
                      acc.at[pl.ds(NS * ROWS_PT, ROWS_TAIL + PAD)])

    plsc.subcore_barrier()

    def pipeline(npairs):
      pltpu.async_copy(y_hbm.at[sidx.at[0]], rows0, semA)

      def body(g, _):
        i0 = 2 * g
        pltpu.async_copy(y_hbm.at[sidx.at[i0 + 1]], rows1, semB)
        pltpu.make_async_copy(y_hbm.at[sidx.at[i0]], rows0, semA).wait()
        pltpu.sync_copy(rows0, acc.at[didx.at[i0]], add=True)

        @pl.when(g < npairs - 1)
        def _():
          pltpu.async_copy(y_hbm.at[sidx.at[i0 + 2]], rows0, semA)

        pltpu.make_async_copy(y_hbm.at[sidx.at[i0 + 1]], rows1, semB).wait()
        pltpu.sync_copy(rows1, acc.at[didx.at[i0 + 1]], add=True)
        return 0

      lax.fori_loop(0, npairs, body, 0)

    @pl.when(wid < TAILW)
    def _():
      for half in range(2):
        base = wid * NCH + half * HCH
        pltpu.sync_copy(src_hbm.at[t, pl.ds(base, HCH)], sidx)
        pltpu.sync_copy(dst_hbm.at[t, pl.ds(base, HCH)], didx)
        pipeline(HCH // 2)

    @pl.when(wid == TAILW)
    def _():
      base = TAILW * NCH
      pltpu.sync_copy(src_hbm.at[t, pl.ds(base, TCHW)],
                      sidx.at[pl.ds(0, TCHW)])
      pltpu.sync_copy(dst_hbm.at[t, pl.ds(base, TCHW)],
                      didx.at[pl.ds(0, TCHW)])
      pipeline(TCHW // 2)

    plsc.subcore_barrier()

    obase = c * T * N + t * N
    for j in range(ROWS_PT // CH):
      pltpu.sync_copy(acc.at[pl.ds(s * ROWS_PT + j * CH, CH)], rows0)
      pltpu.sync_copy(rows0,
                      out_hbm.at[pl.ds(obase + s * ROWS_PT + j * CH, CH)])
    pltpu.sync_copy(acc.at[pl.ds(s * ROWS_PT + (ROWS_PT // CH) * CH, rem)],
                    rows0.at[pl.ds(0, rem)])
    pltpu.sync_copy(rows0.at[pl.ds(0, rem)],
                    out_hbm.at[pl.ds(obase + s * ROWS_PT + (ROWS_PT // CH) * CH,
                                     rem)])

    @pl.when(s == 0)
    def _():
      pltpu.sync_copy(acc.at[pl.ds(NS * ROWS_PT, ROWS_TAIL)],
                      rows1.at[pl.ds(0, ROWS_TAIL)])
      pltpu.sync_copy(rows1.at[pl.ds(0, ROWS_TAIL)],
                      out_hbm.at[pl.ds(obase + NS * ROWS_PT, ROWS_TAIL)])

    plsc.subcore_barrier()


BLK_A = 4000
GRID_A = (T * N) // BLK_A


def _tc_scale_body(x_ref, wt_ref, deg_ref, y_ref):
  deg = deg_ref[:, 0] + deg_ref[:, 1] + 1.0
  dinv = lax.rsqrt(deg)
  xw = jnp.dot(x_ref[...], wt_ref[...], preferred_element_type=jnp.float32)
  y_ref[...] = xw * dinv[:, None]


_tc_scale = pl.pallas_call(
    _tc_scale_body,
    grid=(GRID_A,),
    in_specs=[
        pl.BlockSpec((BLK_A, D), lambda i: (i, 0)),
        pl.BlockSpec((D, H), lambda i: (0, 0)),
        pl.BlockSpec((BLK_A, NC), lambda i: (i, 0)),
    ],
    out_specs=pl.BlockSpec((BLK_A, H), lambda i: (i, 0)),
    out_shape=jax.ShapeDtypeStruct((T * N + PAD, H), jnp.float32),
)


BLK_C = 2000
GRID_C = N // BLK_C


def _tc_gru_body(p_ref, y0_ref, y1_ref, y2_ref, y3_ref, deg_ref, gcn_b_ref,
                 wih_ref, whh_ref, bih_ref, bhh_ref, hw_ref, hb_ref,
                 risk_ref, h_ref):
  h = jnp.zeros((BLK_C, H), dtype=jnp.float32)
  bih = bih_ref[...]
  bhh = bhh_ref[...]
  y_refs = (y0_ref, y1_ref, y2_ref, y3_ref)
  for t in range(T):
    deg = deg_ref[t, :, 0] + deg_ref[t, :, 1] + 1.0
    dinv = lax.rsqrt(deg)
    agg = p_ref[0, t] + p_ref[1, t] + y_refs[t][...]
    xt = jax.nn.relu(agg * dinv[:, None] + gcn_b_ref[...])
    gi = jnp.dot(xt.astype(jnp.bfloat16), wih_ref[...],
                 preferred_element_type=jnp.float32) + bih
    gh = jnp.dot(h.astype(jnp.bfloat16), whh_ref[...],
                 preferred_element_type=jnp.float32) + bhh
    r = jax.nn.sigmoid(gi[:, 0:H] + gh[:, 0:H])
    z = jax.nn.sigmoid(gi[:, H:2 * H] + gh[:, H:2 * H])
    n = jnp.tanh(gi[:, 2 * H:] + r * gh[:, 2 * H:])
    h = (1.0 - z) * n + z * h
  risk = jax.nn.sigmoid(
      jnp.sum(h * hw_ref[...], axis=1, keepdims=True) + hb_ref[0, 0])
  risk_ref[...] = risk
  h_ref[...] = h


_tc_gru = pl.pallas_call(
    _tc_gru_body,
    grid=(GRID_C,),
    in_specs=[
        pl.BlockSpec((NC, T, BLK_C, H), lambda i: (0, 0, i, 0)),
        pl.BlockSpec((BLK_C, H), lambda i: (0 * (N // BLK_C) + i, 0)),
        pl.BlockSpec((BLK_C, H), lambda i: (1 * (N // BLK_C) + i, 0)),
        pl.BlockSpec((BLK_C, H), lambda i: (2 * (N // BLK_C) + i, 0)),
        pl.BlockSpec((BLK_C, H), lambda i: (3 * (N // BLK_C) + i, 0)),
        pl.BlockSpec((T, BLK_C, NC), lambda i: (0, i, 0)),
        pl.BlockSpec((1, H), lambda i: (0, 0)),
        pl.BlockSpec((H, 3 * H), lambda i: (0, 0)),
        pl.BlockSpec((H, 3 * H), lambda i: (0, 0)),
        pl.BlockSpec((1, 3 * H), lambda i: (0, 0)),
        pl.BlockSpec((1, 3 * H), lambda i: (0, 0)),
        pl.BlockSpec((1, H), lambda i: (0, 0)),
        pl.BlockSpec((1, 1), lambda i: (0, 0)),
    ],
    out_specs=[
        pl.BlockSpec((BLK_C, 1), lambda i: (i, 0)),
        pl.BlockSpec((BLK_C, H), lambda i: (i, 0)),
    ],
    out_shape=[
        jax.ShapeDtypeStruct((N, OUT), jnp.float32),
        jax.ShapeDtypeStruct((N, H), jnp.float32),
    ],
)


@functools.lru_cache(maxsize=1)
def _build_sc_kernels():
  mesh = plsc.VectorSubcoreMesh(
      core_axis_name="c", subcore_axis_name="s", num_cores=NC, num_subcores=NS
  )
  deg_kernel = pl.kernel(
      _deg_body,
      out_type=jax.ShapeDtypeStruct((NC * T * N,), jnp.float32),
      mesh=mesh,
      scratch_types=[
          pltpu.VMEM((NCH, CH), jnp.int32),
          pltpu.VMEM((CH,), jnp.float32),
          pltpu.VMEM((DEG_PT,), jnp.float32),
          pltpu.VMEM_SHARED((T * N + PAD,), jnp.float32),
          pltpu.SemaphoreType.DMA,
      ],
  )
  edge_kernel = pl.kernel(
      _edge_body,
      out_type=jax.ShapeDtypeStruct((NC * T * N, H), jnp.float32),
      mesh=mesh,
      scratch_types=[
          pltpu.VMEM((HCH, CH), jnp.int32),
          pltpu.VMEM((HCH, CH), jnp.int32),
          pltpu.VMEM((CH, H), jnp.float32),
          pltpu.VMEM((CH, H), jnp.float32),
          pltpu.VMEM_SHARED((N + PAD, H), jnp.float32),
          pltpu.SemaphoreType.DMA,
          pltpu.SemaphoreType.DMA,
      ],
  )
  return deg_kernel, edge_kernel


def _prep_indices(edge_index_seq):
  toff = (jnp.arange(T, dtype=jnp.int32) * N)[:, None]
  src3 = (edge_index_seq[:, 0, :] + toff).reshape(T, TCH, CH)
  dst3 = edge_index_seq[:, 1, :].reshape(T, TCH, CH)
  didx = (edge_index_seq[:, 1, :] + toff).reshape(T, TCH, CH)
  return src3, dst3, didx


def kernel(x_seq, edge_index_seq, gcn_w, gcn_b, w_ih, w_hh, b_ih, b_hh,
           head_w, head_b):
  deg_kernel, edge_kernel = _build_sc_kernels()
  src3, dst3, didx = _prep_indices(edge_index_seq)

  deg_p = deg_kernel(didx)
  deg2 = deg_p.reshape(NC, T * N).T

  x_flat = x_seq.reshape(T * N, D)
  y = _tc_scale(x_flat, gcn_w.T, deg2)

  p = edge_kernel(y, src3, dst3)

  risk, h_final = _tc_gru(
      p.reshape(NC, T, N, H),
      y, y, y, y,
      deg_p.reshape(NC, T, N).transpose(1, 2, 0),
      gcn_b.reshape(1, H),
      w_ih.T.astype(jnp.bfloat16),
      w_hh.T.astype(jnp.bfloat16),
      b_ih.reshape(1, 3 * H),
      b_hh.reshape(1, 3 * H),
      head_w.reshape(1, H),
      head_b.reshape(1, 1),
  )
  return risk, h_final
